# baseline (speedup 1.0000x reference)
import functools

import jax
import jax.numpy as jnp
from jax import lax
from jax.experimental import pallas as pl
from jax.experimental.pallas import tpu as pltpu
from jax.experimental.pallas import tpu_sc as plsc

NUM_EXPERTS = 64
TOP_K = 8
HIDDEN = 768
BLOCK_T = 4096


def _dense_block(x_ref, w_ref, b_ref, probs_ref, et_ref):
    logits = lax.dot_general(
        w_ref[...], x_ref[...],
        dimension_numbers=(((1,), (1,)), ((), ())),
        preferred_element_type=jnp.float32,
    )
    logits = logits + b_ref[...]
    m = jnp.max(logits, axis=0, keepdims=True)
    e = jnp.exp(logits - m)
    s = jnp.sum(e, axis=0, keepdims=True)
    probs_ref[...] = (e / s).T
    et_ref[...] = e


def _dense_stage(flat_x, gate_weight, bias, n_tokens):
    return pl.pallas_call(
        _dense_block,
        grid=(n_tokens // BLOCK_T,),
        in_specs=[
            pl.BlockSpec((BLOCK_T, HIDDEN), lambda i: (i, 0)),
            pl.BlockSpec((NUM_EXPERTS, HIDDEN), lambda i: (0, 0)),
            pl.BlockSpec((NUM_EXPERTS, 1), lambda i: (0, 0)),
        ],
        out_specs=[
            pl.BlockSpec((BLOCK_T, NUM_EXPERTS), lambda i: (i, 0)),
            pl.BlockSpec((NUM_EXPERTS, BLOCK_T), lambda i: (0, i)),
        ],
        out_shape=[
            jax.ShapeDtypeStruct((n_tokens, NUM_EXPERTS), jnp.float32),
            jax.ShapeDtypeStruct((NUM_EXPERTS, n_tokens), jnp.float32),
        ],
    )(flat_x, gate_weight, bias)


def _make_router_sc(n_tokens):
    info = plsc.get_sparse_core_info()
    nc, ns, nl = info.num_cores, info.num_subcores, info.num_lanes
    nw = nc * ns
    tok_w = n_tokens // nw
    groups = tok_w // nl
    mesh = plsc.VectorSubcoreMesh(core_axis_name="c", subcore_axis_name="s")

    @functools.partial(
        pl.kernel,
        mesh=mesh,
        out_type=[
            jax.ShapeDtypeStruct((TOP_K, n_tokens), jnp.float32),
            jax.ShapeDtypeStruct((TOP_K, n_tokens), jnp.int32),
        ],
        scratch_types=[
            pltpu.VMEM((NUM_EXPERTS * tok_w,), jnp.float32),
            pltpu.VMEM((TOP_K * tok_w,), jnp.float32),
            pltpu.VMEM((TOP_K * tok_w,), jnp.int32),
            pltpu.SemaphoreType.DMA,
        ],
        compiler_params=pltpu.CompilerParams(needs_layout_passes=False),
    )
    def router(et_hbm, tw_hbm, ti_hbm, ev, twv, tiv, sem):
        wid = lax.axis_index("s") * nc + lax.axis_index("c")
        base = wid * tok_w
        copies = [
            pltpu.async_copy(et_hbm.at[ex, pl.ds(base, tok_w)],
                             ev.at[pl.ds(ex * tok_w, tok_w)], sem)
            for ex in range(NUM_EXPERTS)
        ]
        for c in copies:
            c.wait()
        lanes = lax.broadcasted_iota(jnp.int32, (nl,), 0)

        def group_body(g, carry):
            gs = g * nl
            lane_pos = gs + lanes
            ws = jnp.zeros((nl,), jnp.float32)
            picks = []
            for _ in range(TOP_K):
                m = jnp.full((nl,), -1.0, jnp.float32)
                mi = jnp.zeros((nl,), jnp.int32)
                for ex in range(NUM_EXPERTS):
                    v = ev[pl.ds(ex * tok_w + gs, nl)]
                    gt = v > m
                    m = jnp.where(gt, v, m)
                    mi = jnp.where(gt, jnp.full((nl,), ex, jnp.int32), mi)
                plsc.store_scatter(ev, [mi * tok_w + lane_pos],
                                   jnp.full((nl,), -1.0, jnp.float32))
                ws = ws + m
                picks.append((m, mi))
            for k, (m, mi) in enumerate(picks):
                twv[pl.ds(k * tok_w + gs, nl)] = m / ws
                tiv[pl.ds(k * tok_w + gs, nl)] = mi
            return carry

        lax.fori_loop(0, groups, group_body, 0)
        out_copies = [
            pltpu.async_copy(twv.at[pl.ds(k * tok_w, tok_w)],
                             tw_hbm.at[k, pl.ds(base, tok_w)], sem)
            for k in range(TOP_K)
        ] + [
            pltpu.async_copy(tiv.at[pl.ds(k * tok_w, tok_w)],
                             ti_hbm.at[k, pl.ds(base, tok_w)], sem)
            for k in range(TOP_K)
        ]
        for c in out_copies:
            c.wait()

    return router


def kernel(x, gate_weight, expert_bias):
    flat_x = x.reshape(-1, x.shape[-1])
    n_tokens = flat_x.shape[0]
    bias = expert_bias.reshape(NUM_EXPERTS, 1)

    probs, et = _dense_stage(flat_x, gate_weight, bias, n_tokens)
    tw_t, ti_t = _make_router_sc(n_tokens)(et)
    return (tw_t.T, ti_t.T, probs)


# speedup vs baseline: 1.9106x; 1.9106x over previous
import jax
import jax.numpy as jnp
from jax.experimental import pallas as pl

NUM_EXPERTS = 64
TOP_K = 8
HIDDEN = 768
BLOCK_T = 4096


def _router_block(x_ref, w_ref, b_ref, probs_ref, tw_ref, ti_ref):
    logits = jax.lax.dot_general(
        w_ref[...], x_ref[...],
        dimension_numbers=(((1,), (1,)), ((), ())),
        preferred_element_type=jnp.float32,
    )
    logits = logits + b_ref[...]

    m = jnp.max(logits, axis=0, keepdims=True)
    e = jnp.exp(logits - m)
    s = jnp.sum(e, axis=0, keepdims=True)
    probs_ref[...] = (e / s).T

    iota_f = jax.lax.broadcasted_iota(jnp.int32, e.shape, 0).astype(jnp.float32)
    p = e
    rows_w, rows_i = [], []
    wsum = jnp.zeros((1, e.shape[1]), jnp.float32)
    for _ in range(TOP_K):
        cm = jnp.max(p, axis=0, keepdims=True)
        eq = p == cm
        idx = jnp.min(jnp.where(eq, iota_f, 64.0), axis=0, keepdims=True)
        rows_w.append(cm)
        rows_i.append(idx)
        wsum = wsum + cm
        p = jnp.where(eq, -1.0, p)
    tw_ref[...] = jnp.concatenate(rows_w, axis=0) / wsum
    ti_ref[...] = jnp.concatenate(rows_i, axis=0).astype(jnp.int32)


def kernel(x, gate_weight, expert_bias):
    flat_x = x.reshape(-1, x.shape[-1])
    n_tokens = flat_x.shape[0]
    grid = (n_tokens // BLOCK_T,)
    bias = expert_bias.reshape(NUM_EXPERTS, 1)

    probs, tw_t, ti_t = pl.pallas_call(
        _router_block,
        grid=grid,
        in_specs=[
            pl.BlockSpec((BLOCK_T, HIDDEN), lambda i: (i, 0)),
            pl.BlockSpec((NUM_EXPERTS, HIDDEN), lambda i: (0, 0)),
            pl.BlockSpec((NUM_EXPERTS, 1), lambda i: (0, 0)),
        ],
        out_specs=[
            pl.BlockSpec((BLOCK_T, NUM_EXPERTS), lambda i: (i, 0)),
            pl.BlockSpec((TOP_K, BLOCK_T), lambda i: (0, i)),
            pl.BlockSpec((TOP_K, BLOCK_T), lambda i: (0, i)),
        ],
        out_shape=[
            jax.ShapeDtypeStruct((n_tokens, NUM_EXPERTS), jnp.float32),
            jax.ShapeDtypeStruct((TOP_K, n_tokens), jnp.float32),
            jax.ShapeDtypeStruct((TOP_K, n_tokens), jnp.int32),
        ],
    )(flat_x, gate_weight, bias)
    return (tw_t.T, ti_t.T, probs)
